# Initial kernel scaffold; baseline (speedup 1.0000x reference)
import jax
import jax.numpy as jnp
from jax.experimental import pallas as pl


def kernel(x, Wq, bq, Wk, bk, Wv, bv, Wg, bg):
    raise NotImplementedError("write your pallas kernel here")


# speedup vs baseline: 15.9252x; 15.9252x over previous
import math

import jax
import jax.numpy as jnp
from jax.experimental import pallas as pl

_N = 4096
_D = 512
_K = 32
_BR = 256
_INV = 1.0 / math.sqrt(_D)


def _proj_body(x_ref, w3_ref, b3_ref, qkv_ref):
    qkv_ref[...] = jax.lax.dot_general(
        x_ref[...], w3_ref[...], (((1,), (1,)), ((), ())),
        preferred_element_type=jnp.float32) + b3_ref[...]


def _main_body(q_ref, k_ref, x_ref, v_ref, wg1_ref, wg2_ref, bg_ref,
               out_ref, w_ref):
    s = jax.lax.dot_general(
        q_ref[...], k_ref[...], (((1,), (1,)), ((), ())),
        preferred_element_type=jnp.float32) * _INV

    bits = jax.lax.bitcast_convert_type(s, jnp.int32)
    key = jnp.where(bits >= 0, bits, bits ^ jnp.int32(0x7FFFFFFF))

    lo0 = jnp.full((_BR, 1), jnp.iinfo(jnp.int32).min, jnp.int32)
    hi0 = jnp.full((_BR, 1), jnp.iinfo(jnp.int32).max, jnp.int32)

    def body(_, carry):
        lo, hi = carry
        mid = (lo | hi) - ((lo ^ hi) >> 1)
        cnt = jnp.sum((key >= mid).astype(jnp.int32), axis=-1, keepdims=True)
        ge = cnt >= _K
        return jnp.where(ge, mid, lo), jnp.where(ge, hi, mid - 1)

    lo, _ = jax.lax.fori_loop(0, 32, body, (lo0, hi0))

    keep = key >= lo
    m = jnp.max(s, axis=-1, keepdims=True)
    e = jnp.where(keep, jnp.exp(s - m), 0.0)
    z = jnp.sum(e, axis=-1, keepdims=True)
    w = e / z
    w_ref[...] = w

    fused = jax.lax.dot_general(
        w, v_ref[...], (((1,), (0,)), ((), ())),
        preferred_element_type=jnp.float32)
    g = jax.nn.sigmoid(
        jax.lax.dot_general(x_ref[...], wg1_ref[...], (((1,), (1,)), ((), ())),
                            preferred_element_type=jnp.float32)
        + jax.lax.dot_general(fused, wg2_ref[...], (((1,), (1,)), ((), ())),
                              preferred_element_type=jnp.float32)
        + bg_ref[...])
    out_ref[...] = g * x_ref[...] + (1.0 - g) * fused


def kernel(x, Wq, bq, Wk, bk, Wv, bv, Wg, bg):
    nb = _N // _BR
    w3 = jnp.concatenate([Wq, Wk, Wv], axis=0)
    b3 = jnp.concatenate([bq, bk, bv])[None, :]

    qkv = pl.pallas_call(
        _proj_body,
        grid=(nb,),
        in_specs=[pl.BlockSpec((_BR, _D), lambda i: (i, 0)),
                  pl.BlockSpec((3 * _D, _D), lambda i: (0, 0)),
                  pl.BlockSpec((1, 3 * _D), lambda i: (0, 0))],
        out_specs=pl.BlockSpec((_BR, 3 * _D), lambda i: (i, 0)),
        out_shape=jax.ShapeDtypeStruct((_N, 3 * _D), jnp.float32),
    )(x, w3, b3)

    q = qkv[:, :_D]
    k = qkv[:, _D:2 * _D]
    v = qkv[:, 2 * _D:]
    wg1 = Wg[:, :_D]
    wg2 = Wg[:, _D:]

    out, weights = pl.pallas_call(
        _main_body,
        grid=(nb,),
        in_specs=[pl.BlockSpec((_BR, _D), lambda i: (i, 0)),
                  pl.BlockSpec((_N, _D), lambda i: (0, 0)),
                  pl.BlockSpec((_BR, _D), lambda i: (i, 0)),
                  pl.BlockSpec((_N, _D), lambda i: (0, 0)),
                  pl.BlockSpec((_D, _D), lambda i: (0, 0)),
                  pl.BlockSpec((_D, _D), lambda i: (0, 0)),
                  pl.BlockSpec((1, _D), lambda i: (0, 0))],
        out_specs=[pl.BlockSpec((_BR, _D), lambda i: (i, 0)),
                   pl.BlockSpec((_BR, _N), lambda i: (i, 0))],
        out_shape=[jax.ShapeDtypeStruct((_N, _D), jnp.float32),
                   jax.ShapeDtypeStruct((_N, _N), jnp.float32)],
    )(q, k, x, v, wg1, wg2, bg[None, :])
    return out, weights


# speedup vs baseline: 18.6988x; 1.1742x over previous
import math

import jax
import jax.numpy as jnp
from jax.experimental import pallas as pl

_N = 4096
_D = 512
_K = 32
_BR = 256
_NCHUNK = 32
_INV = 1.0 / math.sqrt(_D)


def _key_to_f32(kk):
    return jax.lax.bitcast_convert_type(
        jnp.where(kk >= 0, kk, kk ^ jnp.int32(0x7FFFFFFF)), jnp.float32)


def _f32_to_key(f):
    b = jax.lax.bitcast_convert_type(f, jnp.int32)
    return jnp.where(b >= 0, b, b ^ jnp.int32(0x7FFFFFFF))


def _proj_body(x_ref, w3_ref, b3_ref, qkv_ref):
    qkv_ref[...] = jax.lax.dot_general(
        x_ref[...], w3_ref[...], (((1,), (1,)), ((), ())),
        preferred_element_type=jnp.float32) + b3_ref[...]


def _main_body(q_ref, k_ref, x_ref, v_ref, wg1_ref, wg2_ref, bg_ref,
               out_ref, w_ref):
    dn = (((1,), (1,)), ((), ()))
    s = jax.lax.dot_general(
        q_ref[...], k_ref[...], dn, preferred_element_type=jnp.float32) * _INV

    m = jnp.max(s, axis=-1, keepdims=True)
    cw = _N // _NCHUNK
    lo_f = jnp.max(s[:, :cw], axis=-1, keepdims=True)
    for c in range(1, _NCHUNK):
        lo_f = jnp.minimum(
            lo_f, jnp.max(s[:, c * cw:(c + 1) * cw], axis=-1, keepdims=True))

    lo0 = _f32_to_key(lo_f)
    hi0 = _f32_to_key(m)

    def cond(carry):
        lo, hi = carry
        return jnp.any(lo < hi)

    def body(carry):
        lo, hi = carry
        mid = (lo | hi) - ((lo ^ hi) >> 1)
        mid_f = _key_to_f32(mid)
        cnt = jnp.sum(jnp.where(s >= mid_f, 1.0, 0.0), axis=-1, keepdims=True)
        ge = cnt >= float(_K)
        return jnp.where(ge, mid, lo), jnp.where(ge, hi, mid - 1)

    lo, _ = jax.lax.while_loop(cond, body, (lo0, hi0))
    thr = _key_to_f32(lo)

    e = jnp.where(s >= thr, jnp.exp(s - m), 0.0)
    z = jnp.sum(e, axis=-1, keepdims=True)
    w = e / z
    w_ref[...] = w

    fused = jax.lax.dot_general(
        w, v_ref[...], (((1,), (0,)), ((), ())),
        preferred_element_type=jnp.float32)
    x = x_ref[...]
    g = jax.nn.sigmoid(
        jax.lax.dot_general(x, wg1_ref[...], dn,
                            preferred_element_type=jnp.float32)
        + jax.lax.dot_general(fused, wg2_ref[...], dn,
                              preferred_element_type=jnp.float32)
        + bg_ref[...])
    out_ref[...] = g * x + (1.0 - g) * fused


def kernel(x, Wq, bq, Wk, bk, Wv, bv, Wg, bg):
    nb = _N // _BR
    w3 = jnp.concatenate([Wq, Wk, Wv], axis=0)
    b3 = jnp.concatenate([bq, bk, bv])[None, :]

    qkv = pl.pallas_call(
        _proj_body,
        grid=(nb,),
        in_specs=[pl.BlockSpec((_BR, _D), lambda i: (i, 0)),
                  pl.BlockSpec((3 * _D, _D), lambda i: (0, 0)),
                  pl.BlockSpec((1, 3 * _D), lambda i: (0, 0))],
        out_specs=pl.BlockSpec((_BR, 3 * _D), lambda i: (i, 0)),
        out_shape=jax.ShapeDtypeStruct((_N, 3 * _D), jnp.float32),
    )(x, w3, b3)

    q = qkv[:, :_D]
    k = qkv[:, _D:2 * _D]
    v = qkv[:, 2 * _D:]
    wg1 = Wg[:, :_D]
    wg2 = Wg[:, _D:]

    out, weights = pl.pallas_call(
        _main_body,
        grid=(nb,),
        in_specs=[pl.BlockSpec((_BR, _D), lambda i: (i, 0)),
                  pl.BlockSpec((_N, _D), lambda i: (0, 0)),
                  pl.BlockSpec((_BR, _D), lambda i: (i, 0)),
                  pl.BlockSpec((_N, _D), lambda i: (0, 0)),
                  pl.BlockSpec((_D, _D), lambda i: (0, 0)),
                  pl.BlockSpec((_D, _D), lambda i: (0, 0)),
                  pl.BlockSpec((1, _D), lambda i: (0, 0))],
        out_specs=[pl.BlockSpec((_BR, _D), lambda i: (i, 0)),
                   pl.BlockSpec((_BR, _N), lambda i: (i, 0))],
        out_shape=[jax.ShapeDtypeStruct((_N, _D), jnp.float32),
                   jax.ShapeDtypeStruct((_N, _N), jnp.float32)],
    )(q, k, x, v, wg1, wg2, bg[None, :])
    return out, weights


# speedup vs baseline: 24.2886x; 1.2989x over previous
import math

import jax
import jax.numpy as jnp
from jax.experimental import pallas as pl

_N = 4096
_D = 512
_K = 32
_BR = 256
_NCHUNK = 32
_INV = 1.0 / math.sqrt(_D)


def _key_to_f32(kk):
    return jax.lax.bitcast_convert_type(
        jnp.where(kk >= 0, kk, kk ^ jnp.int32(0x7FFFFFFF)), jnp.float32)


def _f32_to_key(f):
    b = jax.lax.bitcast_convert_type(f, jnp.int32)
    return jnp.where(b >= 0, b, b ^ jnp.int32(0x7FFFFFFF))


def _proj_body(x_ref, w3_ref, b3_ref, qkv_ref):
    qkv_ref[...] = jax.lax.dot_general(
        x_ref[...], w3_ref[...], (((1,), (1,)), ((), ())),
        preferred_element_type=jnp.float32) + b3_ref[...]


def _main_body(q_ref, k_ref, x_ref, v_ref, wg1_ref, wg2_ref, bg_ref,
               out_ref, w_ref):
    dn = (((1,), (1,)), ((), ()))
    s = jax.lax.dot_general(
        q_ref[...], k_ref[...], dn, preferred_element_type=jnp.float32) * _INV

    m = jnp.max(s, axis=-1, keepdims=True)
    cw = _N // _NCHUNK
    lo_f = jnp.max(s[:, :cw], axis=-1, keepdims=True)
    for c in range(1, _NCHUNK):
        lo_f = jnp.minimum(
            lo_f, jnp.max(s[:, c * cw:(c + 1) * cw], axis=-1, keepdims=True))

    lo0 = _f32_to_key(lo_f)
    hi0 = _f32_to_key(m)

    def cond(carry):
        lo, hi = carry
        return jnp.any(lo < hi)

    def body(carry):
        lo, hi = carry
        mid = (lo | hi) - ((lo ^ hi) >> 1)
        mid_f = _key_to_f32(mid)
        cnt = jnp.sum(jnp.where(s >= mid_f, 1.0, 0.0), axis=-1, keepdims=True)
        ge = cnt >= float(_K)
        eq = cnt == float(_K)
        return (jnp.where(ge, mid, lo),
                jnp.where(eq, mid, jnp.where(ge, hi, mid - 1)))

    lo, _ = jax.lax.while_loop(cond, body, (lo0, hi0))
    thr = _key_to_f32(lo)

    e = jnp.where(s >= thr, jnp.exp(s - m), 0.0)
    z = jnp.sum(e, axis=-1, keepdims=True)
    w = e / z
    w_ref[...] = w

    fused = jax.lax.dot_general(
        w, v_ref[...], (((1,), (0,)), ((), ())),
        preferred_element_type=jnp.float32)
    x = x_ref[...]
    g = jax.nn.sigmoid(
        jax.lax.dot_general(x, wg1_ref[...], dn,
                            preferred_element_type=jnp.float32)
        + jax.lax.dot_general(fused, wg2_ref[...], dn,
                              preferred_element_type=jnp.float32)
        + bg_ref[...])
    out_ref[...] = g * x + (1.0 - g) * fused


def kernel(x, Wq, bq, Wk, bk, Wv, bv, Wg, bg):
    nb = _N // _BR
    w3 = jnp.concatenate([Wq, Wk, Wv], axis=0)
    b3 = jnp.concatenate([bq, bk, bv])[None, :]

    qkv = pl.pallas_call(
        _proj_body,
        grid=(nb,),
        in_specs=[pl.BlockSpec((_BR, _D), lambda i: (i, 0)),
                  pl.BlockSpec((3 * _D, _D), lambda i: (0, 0)),
                  pl.BlockSpec((1, 3 * _D), lambda i: (0, 0))],
        out_specs=pl.BlockSpec((_BR, 3 * _D), lambda i: (i, 0)),
        out_shape=jax.ShapeDtypeStruct((_N, 3 * _D), jnp.float32),
    )(x, w3, b3)

    q = qkv[:, :_D]
    k = qkv[:, _D:2 * _D]
    v = qkv[:, 2 * _D:]
    wg1 = Wg[:, :_D]
    wg2 = Wg[:, _D:]

    out, weights = pl.pallas_call(
        _main_body,
        grid=(nb,),
        in_specs=[pl.BlockSpec((_BR, _D), lambda i: (i, 0)),
                  pl.BlockSpec((_N, _D), lambda i: (0, 0)),
                  pl.BlockSpec((_BR, _D), lambda i: (i, 0)),
                  pl.BlockSpec((_N, _D), lambda i: (0, 0)),
                  pl.BlockSpec((_D, _D), lambda i: (0, 0)),
                  pl.BlockSpec((_D, _D), lambda i: (0, 0)),
                  pl.BlockSpec((1, _D), lambda i: (0, 0))],
        out_specs=[pl.BlockSpec((_BR, _D), lambda i: (i, 0)),
                   pl.BlockSpec((_BR, _N), lambda i: (i, 0))],
        out_shape=[jax.ShapeDtypeStruct((_N, _D), jnp.float32),
                   jax.ShapeDtypeStruct((_N, _N), jnp.float32)],
    )(q, k, x, v, wg1, wg2, bg[None, :])
    return out, weights
